# baseline (speedup 1.0000x reference)
import functools

import jax
import jax.numpy as jnp
from jax import lax
from jax.experimental import pallas as pl
from jax.experimental.pallas import tpu as pltpu
from jax.experimental.pallas import tpu_sc as plsc

N = 10000
E = 320000
DIN = 128
DH = 128
DOUT = 64

NC = 2
NS = 16
NW = NC * NS
CHUNK = 128
EPW = 10240
EPAD = NW * EPW
NCHUNK = EPW // CHUNK
NPHASE = 2
HC = NCHUNK // NPHASE
NACC = N + CHUNK
RSTEP = 624
RSPAN = 640

TILE = 1000
NBLK = N // TILE
_PREC = lax.Precision.HIGHEST


NBUF = 2
HGRP = HC // NBUF
CDEP = 4
CGRP = NCHUNK // CDEP


def _sc_agg_body(x_hbm, src_hbm, dst_hbm, zrow_hbm, out_hbm,
                 src_v, dst_v, b0, b1, g0, g1, s0, s1, acc_sh):
    cid = lax.axis_index("c")
    sid = lax.axis_index("s")
    wid = sid * NC + cid
    bufs = (b0, b1)
    gsem = (g0, g1)
    ssem = (s0, s1)

    pltpu.sync_copy(zrow_hbm, acc_sh.at[pl.ds(sid * RSTEP, RSPAN)])
    plsc.subcore_barrier()

    def gather(c, b):
        pltpu.async_copy(x_hbm.at[src_v.at[c]], bufs[b], gsem[b])

    def gather_wait(c, b):
        pltpu.make_async_copy(x_hbm.at[src_v.at[c]], bufs[b], gsem[b]).wait()

    def scatter(c, b):
        pltpu.async_copy(bufs[b], acc_sh.at[dst_v.at[c]], ssem[b], add=True)

    def scatter_wait(c, b):
        pltpu.make_async_copy(bufs[b], acc_sh.at[dst_v.at[c]], ssem[b]).wait()

    def phase(p, carry):
        pltpu.sync_copy(src_hbm.at[wid, pl.ds(p * HC, HC)], src_v)
        pltpu.sync_copy(dst_hbm.at[wid, pl.ds(p * HC, HC)], dst_v)
        for b in range(NBUF):
            gather(b, b)

        def body(g, carry2):
            base = g * NBUF
            for b in range(NBUF):
                gather_wait(base + b, b)
                scatter(base + b, b)
            for b in range(NBUF):
                scatter_wait(base + b, b)
                gather(base + NBUF + b, b)
            return carry2

        lax.fori_loop(0, HGRP - 1, body, 0)
        last = (HGRP - 1) * NBUF
        for b in range(NBUF):
            gather_wait(last + b, b)
            scatter(last + b, b)
        for b in range(NBUF):
            scatter_wait(last + b, b)
        return carry

    lax.fori_loop(0, NPHASE, phase, 0)
    plsc.subcore_barrier()

    row0 = cid * N + sid * RSTEP
    pltpu.sync_copy(acc_sh.at[pl.ds(sid * RSTEP, RSPAN)],
                    out_hbm.at[pl.ds(row0, RSPAN)])


def _make_sc_agg(d):
    mesh = plsc.VectorSubcoreMesh(core_axis_name="c", subcore_axis_name="s")
    out_type = jax.ShapeDtypeStruct((NC * N, d), jnp.float32)
    scratch = (
        [pltpu.VMEM((HC, CHUNK), jnp.int32)] * 2
        + [pltpu.VMEM((CHUNK, d), jnp.float32)] * NBUF
        + [pltpu.SemaphoreType.DMA] * (2 * NBUF)
        + [pltpu.VMEM_SHARED((NACC, d), jnp.float32)]
    )
    return pl.kernel(_sc_agg_body, out_type=out_type, scratch_types=scratch,
                     mesh=mesh)


def _sc_cnt_body(dst_hbm, zcnt_hbm, ones_hbm, cnt_hbm,
                 dst_v, ones_v, s0, s1, s2, s3, cnt_sh):
    cid = lax.axis_index("c")
    sid = lax.axis_index("s")
    wid = sid * NC + cid
    ssem = (s0, s1, s2, s3)

    pltpu.sync_copy(zcnt_hbm, cnt_sh.at[pl.ds(sid * RSTEP, RSPAN)])
    pltpu.sync_copy(ones_hbm, ones_v)
    pltpu.sync_copy(dst_hbm.at[wid], dst_v)
    plsc.subcore_barrier()

    def scatter(c, b):
        pltpu.async_copy(ones_v, cnt_sh.at[dst_v.at[c]], ssem[b], add=True)

    def scatter_wait(c, b):
        pltpu.make_async_copy(ones_v, cnt_sh.at[dst_v.at[c]], ssem[b]).wait()

    for b in range(CDEP):
        scatter(b, b)

    def body(g, carry):
        base = g * CDEP
        for b in range(CDEP):
            scatter_wait(base + b, b)
            scatter(base + CDEP + b, b)
        return carry

    lax.fori_loop(0, CGRP - 1, body, 0)
    last = (CGRP - 1) * CDEP
    for b in range(CDEP):
        scatter_wait(last + b, b)
    plsc.subcore_barrier()

    row0 = cid * N + sid * RSTEP
    pltpu.sync_copy(cnt_sh.at[pl.ds(sid * RSTEP, RSPAN)],
                    cnt_hbm.at[pl.ds(row0, RSPAN)])


def _make_sc_cnt():
    mesh = plsc.VectorSubcoreMesh(core_axis_name="c", subcore_axis_name="s")
    out_type = jax.ShapeDtypeStruct((NC * N, DH), jnp.float32)
    scratch = [
        pltpu.VMEM((NCHUNK, CHUNK), jnp.int32),
        pltpu.VMEM((CHUNK, DH), jnp.float32),
        pltpu.SemaphoreType.DMA, pltpu.SemaphoreType.DMA,
        pltpu.SemaphoreType.DMA, pltpu.SemaphoreType.DMA,
        pltpu.VMEM_SHARED((NACC, DH), jnp.float32),
    ]
    return pl.kernel(_sc_cnt_body, out_type=out_type, scratch_types=scratch,
                     mesh=mesh)


def _dense_stats_body(p0, p1, c0, c1, h_ref, w_ref, r_ref, b_ref,
                      z_ref, st_ref):
    i = pl.program_id(0)
    cnt = c0[:, 0:1] + c1[:, 0:1]
    inv = 1.0 / jnp.maximum(cnt, 1.0)
    t = (p0[...] + p1[...]) * inv
    z = (jnp.dot(t, w_ref[...], preferred_element_type=jnp.float32,
                 precision=_PREC)
         + jnp.dot(h_ref[...], r_ref[...], preferred_element_type=jnp.float32,
                   precision=_PREC)
         + b_ref[...])
    z_ref[...] = z

    @pl.when(i == 0)
    def _():
        st_ref[...] = jnp.zeros_like(st_ref)

    st_ref[0:1, :] += jnp.sum(z, axis=0, keepdims=True)
    st_ref[1:2, :] += jnp.sum(z * z, axis=0, keepdims=True)


def _bn_relu_body(z_ref, st_ref, g_ref, bt_ref, h_ref):
    m = st_ref[0:1, :] / N
    v = st_ref[1:2, :] / N - m * m
    zn = (z_ref[...] - m) * lax.rsqrt(v + 1e-5) * g_ref[...] + bt_ref[...]
    h_ref[...] = jnp.maximum(zn, 0.0)


def _final_body(p0, p1, c0, c1, h_ref, w_ref, r_ref, b_ref, out_ref):
    cnt = c0[:, 0:1] + c1[:, 0:1]
    inv = 1.0 / jnp.maximum(cnt, 1.0)
    t = (p0[...] + p1[...]) * inv
    z = (jnp.dot(t, w_ref[...], preferred_element_type=jnp.float32,
                 precision=_PREC)
         + jnp.dot(h_ref[...], r_ref[...], preferred_element_type=jnp.float32,
                   precision=_PREC)
         + b_ref[...])
    zmax = jnp.max(z, axis=1, keepdims=True)
    lse = jnp.log(jnp.sum(jnp.exp(z - zmax), axis=1, keepdims=True)) + zmax
    out_ref[...] = z - lse


def _part_specs(d):
    return [pl.BlockSpec((TILE, d), lambda i: (i, 0)),
            pl.BlockSpec((TILE, d), lambda i: (i + NBLK, 0))]


def _row_spec(d):
    return pl.BlockSpec((TILE, d), lambda i: (i, 0))


def _fixed_spec(r, c):
    return pl.BlockSpec((r, c), lambda i: (0, 0))


def _dense_stats(parts, cnts, h, W, R, b):
    return pl.pallas_call(
        _dense_stats_body,
        grid=(NBLK,),
        in_specs=_part_specs(DH) + _part_specs(DH)
        + [_row_spec(DH), _fixed_spec(DH, DH), _fixed_spec(DH, DH),
           _fixed_spec(1, DH)],
        out_specs=[_row_spec(DH), _fixed_spec(8, DH)],
        out_shape=[jax.ShapeDtypeStruct((N, DH), jnp.float32),
                   jax.ShapeDtypeStruct((8, DH), jnp.float32)],
    )(parts, parts, cnts, cnts, h, W, R, b.reshape(1, DH))


def _bn_relu(z, st, g, bt):
    return pl.pallas_call(
        _bn_relu_body,
        grid=(NBLK,),
        in_specs=[_row_spec(DH), _fixed_spec(8, DH), _fixed_spec(1, DH),
                  _fixed_spec(1, DH)],
        out_specs=_row_spec(DH),
        out_shape=jax.ShapeDtypeStruct((N, DH), jnp.float32),
    )(z, st, g.reshape(1, DH), bt.reshape(1, DH))


def _final(parts, cnts, h, W3, R3, b3):
    return pl.pallas_call(
        _final_body,
        grid=(NBLK,),
        in_specs=_part_specs(DH) + _part_specs(DH)
        + [_row_spec(DH), _fixed_spec(DH, DOUT), _fixed_spec(DH, DOUT),
           _fixed_spec(1, DOUT)],
        out_specs=_row_spec(DOUT),
        out_shape=jax.ShapeDtypeStruct((N, DOUT), jnp.float32),
    )(parts, parts, cnts, cnts, h, W3, R3, b3.reshape(1, DOUT))


def kernel(x, edge_index, W1, R1, b1, g1, bt1, W2, R2, b2, g2, bt2, W3, R3, b3):
    pad = EPAD - E
    dst = jnp.concatenate(
        [edge_index[0],
         N + (jnp.arange(pad, dtype=jnp.int32) % (NACC - N))]).reshape(
            NW, NCHUNK, CHUNK)
    src = jnp.concatenate(
        [edge_index[1], jnp.zeros((pad,), jnp.int32)]).reshape(
            NW, NCHUNK, CHUNK)
    zrow = jnp.zeros((RSPAN, DH), jnp.float32)
    ones = jnp.ones((CHUNK, DH), jnp.float32)

    agg128 = _make_sc_agg(DH)

    cnts = _make_sc_cnt()(dst, zrow, ones)
    s1 = agg128(x, src, dst, zrow)
    z1, st1 = _dense_stats(s1, cnts, x, W1, R1, b1)
    h1 = _bn_relu(z1, st1, g1, bt1)

    s2 = agg128(h1, src, dst, zrow)
    z2, st2 = _dense_stats(s2, cnts, h1, W2, R2, b2)
    h2 = _bn_relu(z2, st2, g2, bt2)

    s3 = agg128(h2, src, dst, zrow)
    return _final(s3, cnts, h2, W3, R3, b3)


# speedup vs baseline: 2.8357x; 2.8357x over previous
import functools

import jax
import jax.numpy as jnp
from jax import lax
from jax.experimental import pallas as pl
from jax.experimental.pallas import tpu as pltpu
from jax.experimental.pallas import tpu_sc as plsc

N = 10000
E = 320000
DIN = 128
DH = 128
DOUT = 64

NC = 2
NS = 16
NW = NC * NS
CHUNK = 128
EPW = 10240
EPAD = NW * EPW
NCHUNK = EPW // CHUNK
NPHASE = 2
HC = NCHUNK // NPHASE
NACC = N + CHUNK
RSTEP = 624
RSPAN = 640

TILE = 1000
NBLK = N // TILE
_PREC = lax.Precision.HIGHEST


NBUF = 2
HGRP = HC // NBUF
CDEP = 4
CGRP = NCHUNK // CDEP


def _sc_agg_body(x_hbm, src_hbm, dst_hbm, zrow_hbm, out_hbm,
                 src_v, dst_v, b0, b1, g0, g1, s0, s1, acc_sh):
    cid = lax.axis_index("c")
    sid = lax.axis_index("s")
    wid = sid * NC + cid
    bufs = (b0, b1)
    gsem = (g0, g1)
    ssem = (s0, s1)

    pltpu.sync_copy(zrow_hbm, acc_sh.at[pl.ds(sid * RSTEP, RSPAN)])
    plsc.subcore_barrier()

    def gather(c, b):
        pltpu.async_copy(x_hbm.at[src_v.at[c]], bufs[b], gsem[b])

    def gather_wait(c, b):
        pltpu.make_async_copy(x_hbm.at[src_v.at[c]], bufs[b], gsem[b]).wait()

    def scatter(c, b):
        pltpu.async_copy(bufs[b], acc_sh.at[dst_v.at[c]], ssem[b], add=True)

    def scatter_wait(c, b):
        pltpu.make_async_copy(bufs[b], acc_sh.at[dst_v.at[c]], ssem[b]).wait()

    def phase(p, carry):
        pltpu.sync_copy(src_hbm.at[wid, pl.ds(p * HC, HC)], src_v)
        pltpu.sync_copy(dst_hbm.at[wid, pl.ds(p * HC, HC)], dst_v)
        for b in range(NBUF):
            gather(b, b)

        def body(g, carry2):
            base = g * NBUF
            for b in range(NBUF):
                gather_wait(base + b, b)
                scatter(base + b, b)
            for b in range(NBUF):
                scatter_wait(base + b, b)
                gather(base + NBUF + b, b)
            return carry2

        lax.fori_loop(0, HGRP - 1, body, 0)
        last = (HGRP - 1) * NBUF
        for b in range(NBUF):
            gather_wait(last + b, b)
            scatter(last + b, b)
        for b in range(NBUF):
            scatter_wait(last + b, b)
        return carry

    lax.fori_loop(0, NPHASE, phase, 0)
    plsc.subcore_barrier()

    row0 = cid * N + sid * RSTEP
    pltpu.sync_copy(acc_sh.at[pl.ds(sid * RSTEP, RSPAN)],
                    out_hbm.at[pl.ds(row0, RSPAN)])


def _make_sc_agg(d):
    mesh = plsc.VectorSubcoreMesh(core_axis_name="c", subcore_axis_name="s")
    out_type = jax.ShapeDtypeStruct((NC * N, d), jnp.float32)
    scratch = (
        [pltpu.VMEM((HC, CHUNK), jnp.int32)] * 2
        + [pltpu.VMEM((CHUNK, d), jnp.float32)] * NBUF
        + [pltpu.SemaphoreType.DMA] * (2 * NBUF)
        + [pltpu.VMEM_SHARED((NACC, d), jnp.float32)]
    )
    return pl.kernel(_sc_agg_body, out_type=out_type, scratch_types=scratch,
                     mesh=mesh)


def _sc_cnt_body(dst_hbm, zcnt_hbm, ones_hbm, cnt_hbm,
                 dst_v, ones_v, s0, s1, s2, s3, cnt_sh):
    cid = lax.axis_index("c")
    sid = lax.axis_index("s")
    wid = sid * NC + cid
    ssem = (s0, s1, s2, s3)

    pltpu.sync_copy(zcnt_hbm, cnt_sh.at[pl.ds(sid * RSTEP, RSPAN)])
    pltpu.sync_copy(ones_hbm, ones_v)
    pltpu.sync_copy(dst_hbm.at[wid], dst_v)
    plsc.subcore_barrier()

    def scatter(c, b):
        pltpu.async_copy(ones_v, cnt_sh.at[dst_v.at[c]], ssem[b], add=True)

    def scatter_wait(c, b):
        pltpu.make_async_copy(ones_v, cnt_sh.at[dst_v.at[c]], ssem[b]).wait()

    for b in range(CDEP):
        scatter(b, b)

    def body(g, carry):
        base = g * CDEP
        for b in range(CDEP):
            scatter_wait(base + b, b)
            scatter(base + CDEP + b, b)
        return carry

    lax.fori_loop(0, CGRP - 1, body, 0)
    last = (CGRP - 1) * CDEP
    for b in range(CDEP):
        scatter_wait(last + b, b)
    plsc.subcore_barrier()

    row0 = cid * N + sid * RSTEP
    pltpu.sync_copy(cnt_sh.at[pl.ds(sid * RSTEP, RSPAN)],
                    cnt_hbm.at[pl.ds(row0, RSPAN)])


def _make_sc_cnt():
    mesh = plsc.VectorSubcoreMesh(core_axis_name="c", subcore_axis_name="s")
    out_type = jax.ShapeDtypeStruct((NC * N, DH), jnp.float32)
    scratch = [
        pltpu.VMEM((NCHUNK, CHUNK), jnp.int32),
        pltpu.VMEM((CHUNK, DH), jnp.float32),
        pltpu.SemaphoreType.DMA, pltpu.SemaphoreType.DMA,
        pltpu.SemaphoreType.DMA, pltpu.SemaphoreType.DMA,
        pltpu.VMEM_SHARED((NACC, DH), jnp.float32),
    ]
    return pl.kernel(_sc_cnt_body, out_type=out_type, scratch_types=scratch,
                     mesh=mesh)


def _dense_stats_body(p0, p1, c0, c1, h_ref, w_ref, r_ref, b_ref,
                      z_ref, st_ref):
    i = pl.program_id(0)
    cnt = c0[:, 0:1] + c1[:, 0:1]
    inv = 1.0 / jnp.maximum(cnt, 1.0)
    t = (p0[...] + p1[...]) * inv
    z = (jnp.dot(t, w_ref[...], preferred_element_type=jnp.float32,
                 precision=_PREC)
         + jnp.dot(h_ref[...], r_ref[...], preferred_element_type=jnp.float32,
                   precision=_PREC)
         + b_ref[...])
    z_ref[...] = z

    @pl.when(i == 0)
    def _():
        st_ref[...] = jnp.zeros_like(st_ref)

    st_ref[0:1, :] += jnp.sum(z, axis=0, keepdims=True)
    st_ref[1:2, :] += jnp.sum(z * z, axis=0, keepdims=True)


def _bn_relu_body(z_ref, st_ref, g_ref, bt_ref, h_ref):
    m = st_ref[0:1, :] / N
    v = st_ref[1:2, :] / N - m * m
    zn = (z_ref[...] - m) * lax.rsqrt(v + 1e-5) * g_ref[...] + bt_ref[...]
    h_ref[...] = jnp.maximum(zn, 0.0)


def _final_body(p0, p1, c0, c1, h_ref, w_ref, r_ref, b_ref, out_ref):
    cnt = c0[:, 0:1] + c1[:, 0:1]
    inv = 1.0 / jnp.maximum(cnt, 1.0)
    t = (p0[...] + p1[...]) * inv
    z = (jnp.dot(t, w_ref[...], preferred_element_type=jnp.float32,
                 precision=_PREC)
         + jnp.dot(h_ref[...], r_ref[...], preferred_element_type=jnp.float32,
                   precision=_PREC)
         + b_ref[...])
    zmax = jnp.max(z, axis=1, keepdims=True)
    lse = jnp.log(jnp.sum(jnp.exp(z - zmax), axis=1, keepdims=True)) + zmax
    out_ref[...] = z - lse


def _part_specs(d):
    return [pl.BlockSpec((TILE, d), lambda i: (i, 0)),
            pl.BlockSpec((TILE, d), lambda i: (i + NBLK, 0))]


def _row_spec(d):
    return pl.BlockSpec((TILE, d), lambda i: (i, 0))


def _fixed_spec(r, c):
    return pl.BlockSpec((r, c), lambda i: (0, 0))


def _dense_stats(parts, cnts, h, W, R, b):
    return pl.pallas_call(
        _dense_stats_body,
        grid=(NBLK,),
        in_specs=_part_specs(DH) + _part_specs(DH)
        + [_row_spec(DH), _fixed_spec(DH, DH), _fixed_spec(DH, DH),
           _fixed_spec(1, DH)],
        out_specs=[_row_spec(DH), _fixed_spec(8, DH)],
        out_shape=[jax.ShapeDtypeStruct((N, DH), jnp.float32),
                   jax.ShapeDtypeStruct((8, DH), jnp.float32)],
    )(parts, parts, cnts, cnts, h, W, R, b.reshape(1, DH))


def _bn_relu(z, st, g, bt):
    return pl.pallas_call(
        _bn_relu_body,
        grid=(NBLK,),
        in_specs=[_row_spec(DH), _fixed_spec(8, DH), _fixed_spec(1, DH),
                  _fixed_spec(1, DH)],
        out_specs=_row_spec(DH),
        out_shape=jax.ShapeDtypeStruct((N, DH), jnp.float32),
    )(z, st, g.reshape(1, DH), bt.reshape(1, DH))


def _final(parts, cnts, h, W3, R3, b3):
    return pl.pallas_call(
        _final_body,
        grid=(NBLK,),
        in_specs=_part_specs(DH) + _part_specs(DH)
        + [_row_spec(DH), _fixed_spec(DH, DOUT), _fixed_spec(DH, DOUT),
           _fixed_spec(1, DOUT)],
        out_specs=_row_spec(DOUT),
        out_shape=jax.ShapeDtypeStruct((N, DOUT), jnp.float32),
    )(parts, parts, cnts, cnts, h, W3, R3, b3.reshape(1, DOUT))


def kernel(x, edge_index, W1, R1, b1, g1, bt1, W2, R2, b2, g2, bt2, W3, R3, b3):
    pad = EPAD - E
    dst = jnp.concatenate(
        [edge_index[0],
         N + (jnp.arange(pad, dtype=jnp.int32) % (NACC - N))]).reshape(
            NW, NCHUNK, CHUNK)
    src = jnp.concatenate(
        [edge_index[1],
         (jnp.arange(pad, dtype=jnp.int32) * 79) % N]).reshape(
            NW, NCHUNK, CHUNK)
    zrow = jnp.zeros((RSPAN, DH), jnp.float32)
    ones = jnp.ones((CHUNK, DH), jnp.float32)

    agg128 = _make_sc_agg(DH)

    cnts = _make_sc_cnt()(dst, zrow, ones)
    s1 = agg128(x, src, dst, zrow)
    z1, st1 = _dense_stats(s1, cnts, x, W1, R1, b1)
    h1 = _bn_relu(z1, st1, g1, bt1)

    s2 = agg128(h1, src, dst, zrow)
    z2, st2 = _dense_stats(s2, cnts, h1, W2, R2, b2)
    h2 = _bn_relu(z2, st2, g2, bt2)

    s3 = agg128(h2, src, dst, zrow)
    return _final(s3, cnts, h2, W3, R3, b3)


# speedup vs baseline: 3.2208x; 1.1358x over previous
import functools

import jax
import jax.numpy as jnp
from jax import lax
from jax.experimental import pallas as pl
from jax.experimental.pallas import tpu as pltpu
from jax.experimental.pallas import tpu_sc as plsc

N = 10000
E = 320000
DIN = 128
DH = 128
DOUT = 64

NC = 2
NS = 16
NW = NC * NS
CHUNK = 80
EPW = 10240
EPAD = NW * EPW
NCHUNK = EPW // CHUNK
NPHASE = 4
HC = NCHUNK // NPHASE
NACC = N + CHUNK
RSTEP = 624
RSPAN = 640

TILE = 1000
NBLK = N // TILE
_PREC = lax.Precision.HIGHEST


NBUF = 4
HGRP = HC // NBUF
CDEP = 4
CGRP = NCHUNK // CDEP


def _sc_agg_body(x_hbm, src_hbm, dst_hbm, zrow_hbm, out_hbm,
                 src_v, dst_v, b0, b1, b2, b3,
                 g0, g1, g2, g3, s0, s1, s2, s3, acc_sh):
    cid = lax.axis_index("c")
    sid = lax.axis_index("s")
    wid = sid * NC + cid
    bufs = (b0, b1, b2, b3)
    gsem = (g0, g1, g2, g3)
    ssem = (s0, s1, s2, s3)

    pltpu.sync_copy(zrow_hbm, acc_sh.at[pl.ds(sid * RSTEP, RSPAN)])
    plsc.subcore_barrier()

    def gather(c, b):
        pltpu.async_copy(x_hbm.at[src_v.at[c]], bufs[b], gsem[b])

    def gather_wait(c, b):
        pltpu.make_async_copy(x_hbm.at[src_v.at[c]], bufs[b], gsem[b]).wait()

    def scatter(c, b):
        pltpu.async_copy(bufs[b], acc_sh.at[dst_v.at[c]], ssem[b], add=True)

    def scatter_wait(c, b):
        pltpu.make_async_copy(bufs[b], acc_sh.at[dst_v.at[c]], ssem[b]).wait()

    def phase(p, carry):
        pltpu.sync_copy(src_hbm.at[wid, pl.ds(p * HC, HC)], src_v)
        pltpu.sync_copy(dst_hbm.at[wid, pl.ds(p * HC, HC)], dst_v)
        for b in range(NBUF):
            gather(b, b)

        def body(g, carry2):
            base = g * NBUF
            for b in range(NBUF):
                gather_wait(base + b, b)
                scatter(base + b, b)
            for b in range(NBUF):
                scatter_wait(base + b, b)
                gather(base + NBUF + b, b)
            return carry2

        lax.fori_loop(0, HGRP - 1, body, 0)
        last = (HGRP - 1) * NBUF
        for b in range(NBUF):
            gather_wait(last + b, b)
            scatter(last + b, b)
        for b in range(NBUF):
            scatter_wait(last + b, b)
        return carry

    lax.fori_loop(0, NPHASE, phase, 0)
    plsc.subcore_barrier()

    row0 = cid * N + sid * RSTEP
    pltpu.sync_copy(acc_sh.at[pl.ds(sid * RSTEP, RSPAN)],
                    out_hbm.at[pl.ds(row0, RSPAN)])


def _make_sc_agg(d):
    mesh = plsc.VectorSubcoreMesh(core_axis_name="c", subcore_axis_name="s")
    out_type = jax.ShapeDtypeStruct((NC * N, d), jnp.float32)
    scratch = (
        [pltpu.VMEM((HC, CHUNK), jnp.int32)] * 2
        + [pltpu.VMEM((CHUNK, d), jnp.float32)] * NBUF
        + [pltpu.SemaphoreType.DMA] * (2 * NBUF)
        + [pltpu.VMEM_SHARED((NACC, d), jnp.float32)]
    )
    return pl.kernel(_sc_agg_body, out_type=out_type, scratch_types=scratch,
                     mesh=mesh)


def _sc_cnt_body(dst_hbm, zcnt_hbm, ones_hbm, cnt_hbm,
                 dst_v, ones_v, s0, s1, s2, s3, cnt_sh):
    cid = lax.axis_index("c")
    sid = lax.axis_index("s")
    wid = sid * NC + cid
    ssem = (s0, s1, s2, s3)

    pltpu.sync_copy(zcnt_hbm, cnt_sh.at[pl.ds(sid * RSTEP, RSPAN)])
    pltpu.sync_copy(ones_hbm, ones_v)
    pltpu.sync_copy(dst_hbm.at[wid], dst_v)
    plsc.subcore_barrier()

    def scatter(c, b):
        pltpu.async_copy(ones_v, cnt_sh.at[dst_v.at[c]], ssem[b], add=True)

    def scatter_wait(c, b):
        pltpu.make_async_copy(ones_v, cnt_sh.at[dst_v.at[c]], ssem[b]).wait()

    for b in range(CDEP):
        scatter(b, b)

    def body(g, carry):
        base = g * CDEP
        for b in range(CDEP):
            scatter_wait(base + b, b)
            scatter(base + CDEP + b, b)
        return carry

    lax.fori_loop(0, CGRP - 1, body, 0)
    last = (CGRP - 1) * CDEP
    for b in range(CDEP):
        scatter_wait(last + b, b)
    plsc.subcore_barrier()

    row0 = cid * N + sid * RSTEP
    pltpu.sync_copy(cnt_sh.at[pl.ds(sid * RSTEP, RSPAN)],
                    cnt_hbm.at[pl.ds(row0, RSPAN)])


def _make_sc_cnt():
    mesh = plsc.VectorSubcoreMesh(core_axis_name="c", subcore_axis_name="s")
    out_type = jax.ShapeDtypeStruct((NC * N, DH), jnp.float32)
    scratch = [
        pltpu.VMEM((NCHUNK, CHUNK), jnp.int32),
        pltpu.VMEM((CHUNK, DH), jnp.float32),
        pltpu.SemaphoreType.DMA, pltpu.SemaphoreType.DMA,
        pltpu.SemaphoreType.DMA, pltpu.SemaphoreType.DMA,
        pltpu.VMEM_SHARED((NACC, DH), jnp.float32),
    ]
    return pl.kernel(_sc_cnt_body, out_type=out_type, scratch_types=scratch,
                     mesh=mesh)


def _dense_stats_body(p0, p1, c0, c1, h_ref, w_ref, r_ref, b_ref,
                      z_ref, st_ref):
    i = pl.program_id(0)
    cnt = c0[:, 0:1] + c1[:, 0:1]
    inv = 1.0 / jnp.maximum(cnt, 1.0)
    t = (p0[...] + p1[...]) * inv
    z = (jnp.dot(t, w_ref[...], preferred_element_type=jnp.float32,
                 precision=_PREC)
         + jnp.dot(h_ref[...], r_ref[...], preferred_element_type=jnp.float32,
                   precision=_PREC)
         + b_ref[...])
    z_ref[...] = z

    @pl.when(i == 0)
    def _():
        st_ref[...] = jnp.zeros_like(st_ref)

    st_ref[0:1, :] += jnp.sum(z, axis=0, keepdims=True)
    st_ref[1:2, :] += jnp.sum(z * z, axis=0, keepdims=True)


def _bn_relu_body(z_ref, st_ref, g_ref, bt_ref, h_ref):
    m = st_ref[0:1, :] / N
    v = st_ref[1:2, :] / N - m * m
    zn = (z_ref[...] - m) * lax.rsqrt(v + 1e-5) * g_ref[...] + bt_ref[...]
    h_ref[...] = jnp.maximum(zn, 0.0)


def _final_body(p0, p1, c0, c1, h_ref, w_ref, r_ref, b_ref, out_ref):
    cnt = c0[:, 0:1] + c1[:, 0:1]
    inv = 1.0 / jnp.maximum(cnt, 1.0)
    t = (p0[...] + p1[...]) * inv
    z = (jnp.dot(t, w_ref[...], preferred_element_type=jnp.float32,
                 precision=_PREC)
         + jnp.dot(h_ref[...], r_ref[...], preferred_element_type=jnp.float32,
                   precision=_PREC)
         + b_ref[...])
    zmax = jnp.max(z, axis=1, keepdims=True)
    lse = jnp.log(jnp.sum(jnp.exp(z - zmax), axis=1, keepdims=True)) + zmax
    out_ref[...] = z - lse


def _part_specs(d):
    return [pl.BlockSpec((TILE, d), lambda i: (i, 0)),
            pl.BlockSpec((TILE, d), lambda i: (i + NBLK, 0))]


def _row_spec(d):
    return pl.BlockSpec((TILE, d), lambda i: (i, 0))


def _fixed_spec(r, c):
    return pl.BlockSpec((r, c), lambda i: (0, 0))


def _dense_stats(parts, cnts, h, W, R, b):
    return pl.pallas_call(
        _dense_stats_body,
        grid=(NBLK,),
        in_specs=_part_specs(DH) + _part_specs(DH)
        + [_row_spec(DH), _fixed_spec(DH, DH), _fixed_spec(DH, DH),
           _fixed_spec(1, DH)],
        out_specs=[_row_spec(DH), _fixed_spec(8, DH)],
        out_shape=[jax.ShapeDtypeStruct((N, DH), jnp.float32),
                   jax.ShapeDtypeStruct((8, DH), jnp.float32)],
    )(parts, parts, cnts, cnts, h, W, R, b.reshape(1, DH))


def _bn_relu(z, st, g, bt):
    return pl.pallas_call(
        _bn_relu_body,
        grid=(NBLK,),
        in_specs=[_row_spec(DH), _fixed_spec(8, DH), _fixed_spec(1, DH),
                  _fixed_spec(1, DH)],
        out_specs=_row_spec(DH),
        out_shape=jax.ShapeDtypeStruct((N, DH), jnp.float32),
    )(z, st, g.reshape(1, DH), bt.reshape(1, DH))


def _final(parts, cnts, h, W3, R3, b3):
    return pl.pallas_call(
        _final_body,
        grid=(NBLK,),
        in_specs=_part_specs(DH) + _part_specs(DH)
        + [_row_spec(DH), _fixed_spec(DH, DOUT), _fixed_spec(DH, DOUT),
           _fixed_spec(1, DOUT)],
        out_specs=_row_spec(DOUT),
        out_shape=jax.ShapeDtypeStruct((N, DOUT), jnp.float32),
    )(parts, parts, cnts, cnts, h, W3, R3, b3.reshape(1, DOUT))


def kernel(x, edge_index, W1, R1, b1, g1, bt1, W2, R2, b2, g2, bt2, W3, R3, b3):
    pad = EPAD - E
    dst = jnp.concatenate(
        [edge_index[0],
         N + (jnp.arange(pad, dtype=jnp.int32) % (NACC - N))]).reshape(
            NW, NCHUNK, CHUNK)
    src = jnp.concatenate(
        [edge_index[1],
         (jnp.arange(pad, dtype=jnp.int32) * 79) % N]).reshape(
            NW, NCHUNK, CHUNK)
    zrow = jnp.zeros((RSPAN, DH), jnp.float32)
    ones = jnp.ones((CHUNK, DH), jnp.float32)

    agg128 = _make_sc_agg(DH)

    cnts = _make_sc_cnt()(dst, zrow, ones)
    s1 = agg128(x, src, dst, zrow)
    z1, st1 = _dense_stats(s1, cnts, x, W1, R1, b1)
    h1 = _bn_relu(z1, st1, g1, bt1)

    s2 = agg128(h1, src, dst, zrow)
    z2, st2 = _dense_stats(s2, cnts, h1, W2, R2, b2)
    h2 = _bn_relu(z2, st2, g2, bt2)

    s3 = agg128(h2, src, dst, zrow)
    return _final(s3, cnts, h2, W3, R3, b3)


# speedup vs baseline: 3.2805x; 1.0185x over previous
import functools

import jax
import jax.numpy as jnp
from jax import lax
from jax.experimental import pallas as pl
from jax.experimental.pallas import tpu as pltpu
from jax.experimental.pallas import tpu_sc as plsc

N = 10000
E = 320000
DIN = 128
DH = 128
DOUT = 64

NC = 2
NS = 16
NW = NC * NS
CHUNK = 80
EPW = 10240
EPAD = NW * EPW
NCHUNK = EPW // CHUNK
NPHASE = 4
HC = NCHUNK // NPHASE
NACC = N + CHUNK
RSTEP = 624
RSPAN = 640

TILE = 1000
NBLK = N // TILE
_PREC = lax.Precision.HIGHEST


NBUF = 4
HGRP = HC // NBUF
CDEP = 4
CGRP = NCHUNK // CDEP


def _sc_agg_body(x_hbm, src_hbm, dst_hbm, zrow_hbm, out_hbm,
                 src_v, dst_v, b0, b1, b2, b3,
                 g0, g1, g2, g3, s0, s1, s2, s3, acc_sh):
    cid = lax.axis_index("c")
    sid = lax.axis_index("s")
    wid = sid * NC + cid
    bufs = (b0, b1, b2, b3)
    gsem = (g0, g1, g2, g3)
    ssem = (s0, s1, s2, s3)

    pltpu.sync_copy(zrow_hbm, acc_sh.at[pl.ds(sid * RSTEP, RSPAN)])
    plsc.subcore_barrier()

    def gather(c, b):
        pltpu.async_copy(x_hbm.at[src_v.at[c]], bufs[b], gsem[b])

    def gather_wait(c, b):
        pltpu.make_async_copy(x_hbm.at[src_v.at[c]], bufs[b], gsem[b]).wait()

    def scatter(c, b):
        pltpu.async_copy(bufs[b], acc_sh.at[dst_v.at[c]], ssem[b], add=True)

    def scatter_wait(c, b):
        pltpu.make_async_copy(bufs[b], acc_sh.at[dst_v.at[c]], ssem[b]).wait()

    def phase(p, carry):
        pltpu.sync_copy(src_hbm.at[wid, pl.ds(p * HC, HC)], src_v)
        pltpu.sync_copy(dst_hbm.at[wid, pl.ds(p * HC, HC)], dst_v)
        for b in range(NBUF):
            gather(b, b)

        def body(g, carry2):
            base = g * NBUF
            for b in range(NBUF):
                gather_wait(base + b, b)
                scatter(base + b, b)
            for b in range(NBUF):
                scatter_wait(base + b, b)
                gather(base + NBUF + b, b)
            return carry2

        lax.fori_loop(0, HGRP - 1, body, 0)
        last = (HGRP - 1) * NBUF
        for b in range(NBUF):
            gather_wait(last + b, b)
            scatter(last + b, b)
        for b in range(NBUF):
            scatter_wait(last + b, b)
        return carry

    lax.fori_loop(0, NPHASE, phase, 0)
    plsc.subcore_barrier()

    row0 = cid * N + sid * RSTEP
    pltpu.sync_copy(acc_sh.at[pl.ds(sid * RSTEP, RSPAN)],
                    out_hbm.at[pl.ds(row0, RSPAN)])


def _make_sc_agg(d):
    mesh = plsc.VectorSubcoreMesh(core_axis_name="c", subcore_axis_name="s")
    out_type = jax.ShapeDtypeStruct((NC * N, d), jnp.float32)
    scratch = (
        [pltpu.VMEM((HC, CHUNK), jnp.int32)] * 2
        + [pltpu.VMEM((CHUNK, d), jnp.float32)] * NBUF
        + [pltpu.SemaphoreType.DMA] * (2 * NBUF)
        + [pltpu.VMEM_SHARED((NACC, d), jnp.float32)]
    )
    return pl.kernel(_sc_agg_body, out_type=out_type, scratch_types=scratch,
                     mesh=mesh)


def _sc_cnt_body(dst_hbm, zcnt_hbm, ones_hbm, cnt_hbm,
                 dst_v, ones_v, s0, s1, s2, s3, cnt_sh):
    cid = lax.axis_index("c")
    sid = lax.axis_index("s")
    wid = sid * NC + cid
    ssem = (s0, s1, s2, s3)

    pltpu.sync_copy(zcnt_hbm, cnt_sh.at[pl.ds(sid * RSTEP, RSPAN)])
    pltpu.sync_copy(ones_hbm, ones_v)
    pltpu.sync_copy(dst_hbm.at[wid], dst_v)
    plsc.subcore_barrier()

    def scatter(c, b):
        pltpu.async_copy(ones_v, cnt_sh.at[dst_v.at[c]], ssem[b], add=True)

    def scatter_wait(c, b):
        pltpu.make_async_copy(ones_v, cnt_sh.at[dst_v.at[c]], ssem[b]).wait()

    for b in range(CDEP):
        scatter(b, b)

    def body(g, carry):
        base = g * CDEP
        for b in range(CDEP):
            scatter_wait(base + b, b)
            scatter(base + CDEP + b, b)
        return carry

    lax.fori_loop(0, CGRP - 1, body, 0)
    last = (CGRP - 1) * CDEP
    for b in range(CDEP):
        scatter_wait(last + b, b)
    plsc.subcore_barrier()

    row0 = cid * N + sid * RSTEP
    pltpu.sync_copy(cnt_sh.at[pl.ds(sid * RSTEP, RSPAN)],
                    cnt_hbm.at[pl.ds(row0, RSPAN)])


def _make_sc_cnt():
    mesh = plsc.VectorSubcoreMesh(core_axis_name="c", subcore_axis_name="s")
    out_type = jax.ShapeDtypeStruct((NC * N, DH), jnp.float32)
    scratch = [
        pltpu.VMEM((NCHUNK, CHUNK), jnp.int32),
        pltpu.VMEM((CHUNK, DH), jnp.float32),
        pltpu.SemaphoreType.DMA, pltpu.SemaphoreType.DMA,
        pltpu.SemaphoreType.DMA, pltpu.SemaphoreType.DMA,
        pltpu.VMEM_SHARED((NACC, DH), jnp.float32),
    ]
    return pl.kernel(_sc_cnt_body, out_type=out_type, scratch_types=scratch,
                     mesh=mesh)


def _layer_body(p0, p1, c0, c1, h_ref, w_ref, r_ref, b_ref, g_ref, bt_ref,
                out_ref, z_scr, st_scr):
    ph = pl.program_id(0)
    i = pl.program_id(1)

    @pl.when(ph == 0)
    def _():
        cnt = c0[:, 0:1] + c1[:, 0:1]
        inv = 1.0 / jnp.maximum(cnt, 1.0)
        t = (p0[...] + p1[...]) * inv
        z = (jnp.dot(t, w_ref[...], preferred_element_type=jnp.float32,
                     precision=_PREC)
             + jnp.dot(h_ref[...], r_ref[...],
                       preferred_element_type=jnp.float32, precision=_PREC)
             + b_ref[...])
        z_scr[pl.ds(i * TILE, TILE), :] = z

        @pl.when(i == 0)
        def _():
            st_scr[...] = jnp.zeros_like(st_scr)

        st_scr[0:1, :] += jnp.sum(z, axis=0, keepdims=True)
        st_scr[1:2, :] += jnp.sum(z * z, axis=0, keepdims=True)

    @pl.when(ph == 1)
    def _():
        m = st_scr[0:1, :] / N
        v = st_scr[1:2, :] / N - m * m
        z = z_scr[pl.ds(i * TILE, TILE), :]
        zn = (z - m) * lax.rsqrt(v + 1e-5) * g_ref[...] + bt_ref[...]
        out_ref[...] = jnp.maximum(zn, 0.0)


def _final_body(p0, p1, c0, c1, h_ref, w_ref, r_ref, b_ref, out_ref):
    cnt = c0[:, 0:1] + c1[:, 0:1]
    inv = 1.0 / jnp.maximum(cnt, 1.0)
    t = (p0[...] + p1[...]) * inv
    z = (jnp.dot(t, w_ref[...], preferred_element_type=jnp.float32,
                 precision=_PREC)
         + jnp.dot(h_ref[...], r_ref[...], preferred_element_type=jnp.float32,
                   precision=_PREC)
         + b_ref[...])
    zmax = jnp.max(z, axis=1, keepdims=True)
    lse = jnp.log(jnp.sum(jnp.exp(z - zmax), axis=1, keepdims=True)) + zmax
    out_ref[...] = z - lse


def _part_specs2(d):
    return [pl.BlockSpec((TILE, d), lambda ph, i: (i * (1 - ph), 0)),
            pl.BlockSpec((TILE, d), lambda ph, i: (i * (1 - ph) + NBLK, 0))]


def _fixed_spec2(r, c):
    return pl.BlockSpec((r, c), lambda ph, i: (0, 0))


def _layer(parts, cnts, h, W, R, b, g, bt):
    return pl.pallas_call(
        _layer_body,
        grid=(2, NBLK),
        in_specs=_part_specs2(DH) + _part_specs2(DH)
        + [pl.BlockSpec((TILE, DH), lambda ph, i: (i * (1 - ph), 0)),
           _fixed_spec2(DH, DH), _fixed_spec2(DH, DH), _fixed_spec2(1, DH),
           _fixed_spec2(1, DH), _fixed_spec2(1, DH)],
        out_specs=pl.BlockSpec((TILE, DH), lambda ph, i: (i * ph, 0)),
        out_shape=jax.ShapeDtypeStruct((N, DH), jnp.float32),
        scratch_shapes=[pltpu.VMEM((N, DH), jnp.float32),
                        pltpu.VMEM((8, DH), jnp.float32)],
    )(parts, parts, cnts, cnts, h, W, R, b.reshape(1, DH),
      g.reshape(1, DH), bt.reshape(1, DH))


def _part_specs(d):
    return [pl.BlockSpec((TILE, d), lambda i: (i, 0)),
            pl.BlockSpec((TILE, d), lambda i: (i + NBLK, 0))]


def _row_spec(d):
    return pl.BlockSpec((TILE, d), lambda i: (i, 0))


def _fixed_spec(r, c):
    return pl.BlockSpec((r, c), lambda i: (0, 0))


def _final(parts, cnts, h, W3, R3, b3):
    return pl.pallas_call(
        _final_body,
        grid=(NBLK,),
        in_specs=_part_specs(DH) + _part_specs(DH)
        + [_row_spec(DH), _fixed_spec(DH, DOUT), _fixed_spec(DH, DOUT),
           _fixed_spec(1, DOUT)],
        out_specs=_row_spec(DOUT),
        out_shape=jax.ShapeDtypeStruct((N, DOUT), jnp.float32),
    )(parts, parts, cnts, cnts, h, W3, R3, b3.reshape(1, DOUT))


def kernel(x, edge_index, W1, R1, b1, g1, bt1, W2, R2, b2, g2, bt2, W3, R3, b3):
    pad = EPAD - E
    dst = jnp.concatenate(
        [edge_index[0],
         N + (jnp.arange(pad, dtype=jnp.int32) % (NACC - N))]).reshape(
            NW, NCHUNK, CHUNK)
    src = jnp.concatenate(
        [edge_index[1],
         (jnp.arange(pad, dtype=jnp.int32) * 79) % N]).reshape(
            NW, NCHUNK, CHUNK)
    zrow = jnp.zeros((RSPAN, DH), jnp.float32)
    ones = jnp.ones((CHUNK, DH), jnp.float32)

    agg128 = _make_sc_agg(DH)

    cnts = _make_sc_cnt()(dst, zrow, ones)
    s1 = agg128(x, src, dst, zrow)
    h1 = _layer(s1, cnts, x, W1, R1, b1, g1, bt1)

    s2 = agg128(h1, src, dst, zrow)
    h2 = _layer(s2, cnts, h1, W2, R2, b2, g2, bt2)

    s3 = agg128(h2, src, dst, zrow)
    return _final(s3, cnts, h2, W3, R3, b3)
